# Initial kernel scaffold; baseline (speedup 1.0000x reference)
import jax
import jax.numpy as jnp
from jax.experimental import pallas as pl


def kernel(h, e, edge_index, params):
    raise NotImplementedError("write your pallas kernel here")


# speedup vs baseline: 6.9539x; 6.9539x over previous
import functools

import jax
import jax.numpy as jnp
from jax import lax
from jax.experimental import pallas as pl
from jax.experimental.pallas import tpu as pltpu
from jax.experimental.pallas import tpu_sc as plsc

N = 10000
E = 160000
D = 128
H = 8
DH = D // H

NC = 2
NS = 16
NW = NC * NS
CH = 128
NG = CH // 16
NCHUNK = E // CH
CHUNKS_PER_W = -(-NCHUNK // NW)
NPAD = 10240
ROWS_PER_TILE = NPAD // NS
NZROW = NPAD // 16
ZROWS_PER_TILE = NZROW // NS


def _ln_tc(x, g, b):
    m = jnp.mean(x, axis=-1, keepdims=True)
    v = jnp.mean((x - m) * (x - m), axis=-1, keepdims=True)
    return (x - m) / jnp.sqrt(v + 1e-5) * g + b


def _qkv_body(h_ref, wq_ref, wk_ref, wv_ref, q_ref, k_ref, v_ref):
    hb = h_ref[...]
    q_ref[...] = jnp.dot(hb, wq_ref[...], preferred_element_type=jnp.float32)
    k_ref[...] = jnp.dot(hb, wk_ref[...], preferred_element_type=jnp.float32)
    v_ref[...] = jnp.dot(hb, wv_ref[...], preferred_element_type=jnp.float32)


def _tc_qkv(h, wq, wk, wv):
    bn = 2000
    w_spec = pl.BlockSpec((D, D), lambda i: (0, 0))
    r_spec = pl.BlockSpec((bn, D), lambda i: (i, 0))
    return pl.pallas_call(
        _qkv_body,
        grid=(N // bn,),
        in_specs=[r_spec, w_spec, w_spec, w_spec],
        out_specs=[r_spec, r_spec, r_spec],
        out_shape=[jax.ShapeDtypeStruct((N, D), jnp.float32)] * 3,
    )(h, wq, wk, wv)


def _pesum_body(e_ref, we_ref, ps_ref):
    pe = jnp.dot(e_ref[...], we_ref[...], preferred_element_type=jnp.float32)
    r = lax.broadcasted_iota(jnp.int32, (D, H), 0)
    c = lax.broadcasted_iota(jnp.int32, (D, H), 1)
    sel = (r // DH == c).astype(jnp.float32)
    ps_ref[...] = jnp.dot(pe, sel, preferred_element_type=jnp.float32)


def _tc_pesum(e, we):
    be = 2000
    return pl.pallas_call(
        _pesum_body,
        grid=(E // be,),
        in_specs=[pl.BlockSpec((be, D), lambda i: (i, 0)),
                  pl.BlockSpec((D, D), lambda i: (0, 0))],
        out_specs=pl.BlockSpec((be, H), lambda i: (i, 0)),
        out_shape=jax.ShapeDtypeStruct((E, H), jnp.float32),
    )(e, we)


NH = NPAD // 2
AWROWS = NH + CH
AWZ_PER_TILE = AWROWS // NS
RH_PER_TILE = NH // NS
CHP = CH // 16
EP = E // 16


def _sc_attention(qh, kh, vh, src, dst, pes2):
    mesh = plsc.VectorSubcoreMesh(
        core_axis_name="c", subcore_axis_name="s",
        num_cores=NC, num_subcores=NS)

    @functools.partial(
        pl.kernel,
        out_type=[
            jax.ShapeDtypeStruct((EP, D), jnp.float32),
            jax.ShapeDtypeStruct((EP, D), jnp.float32),
            jax.ShapeDtypeStruct((NC, 2, NH, D), jnp.float32),
            jax.ShapeDtypeStruct((NC, NZROW, D), jnp.float32),
        ],
        mesh=mesh,
        compiler_params=pltpu.CompilerParams(needs_layout_passes=False),
        scratch_types=[
            pltpu.VMEM((CH, D), jnp.float32),
            pltpu.VMEM((CH, D), jnp.float32),
            pltpu.VMEM((CHP, D), jnp.float32),
            pltpu.VMEM((CHP, D), jnp.float32),
            pltpu.VMEM((CHP, D), jnp.float32),
            pltpu.VMEM((CH, D), jnp.float32),
            pltpu.VMEM((CH, D), jnp.float32),
            pltpu.VMEM((CH,), jnp.int32),
            pltpu.VMEM((CH,), jnp.int32),
            pltpu.VMEM_SHARED((AWROWS, D), jnp.float32),
            pltpu.VMEM_SHARED((NZROW, D), jnp.float32),
            pltpu.SemaphoreType.DMA,
            pltpu.SemaphoreType.DMA,
        ],
    )
    def sc_kern(qh_hbm, kh_hbm, vh_hbm, src_hbm, dst_hbm, pes_hbm,
                sraw_hbm, sexp_hbm, accwv_hbm, accz_hbm,
                kbuf, qbuf, pesbuf, srawbuf, sxbuf, crow, zrowbuf,
                srcbuf, dstbuf,
                accwv, accz, sem0, sem1):
        cid = lax.axis_index("c")
        sid = lax.axis_index("s")
        wid = sid * NC + cid
        lane = lax.broadcasted_iota(jnp.int32, (16,), 0)
        zeros16 = jnp.zeros((16,), jnp.float32)

        def zero_accwv_slice():
            r0 = sid * AWZ_PER_TILE
            pltpu.sync_copy(zrowbuf, accwv.at[pl.ds(r0, CH)])
            pltpu.sync_copy(zrowbuf, accwv.at[pl.ds(r0 + CH, CH)])
            pltpu.sync_copy(zrowbuf.at[pl.ds(0, AWZ_PER_TILE - 2 * CH)],
                            accwv.at[pl.ds(r0 + 2 * CH, AWZ_PER_TILE - 2 * CH)])

        def read_out_accwv(p):
            r0 = sid * RH_PER_TILE
            pltpu.sync_copy(accwv.at[pl.ds(r0, CH)],
                            accwv_hbm.at[cid, p, pl.ds(r0, CH)])
            pltpu.sync_copy(accwv.at[pl.ds(r0 + CH, CH)],
                            accwv_hbm.at[cid, p, pl.ds(r0 + CH, CH)])
            rem = RH_PER_TILE - 2 * CH
            pltpu.sync_copy(accwv.at[pl.ds(r0 + 2 * CH, rem)],
                            accwv_hbm.at[cid, p, pl.ds(r0 + 2 * CH, rem)])

        def zrow_init(i, carry):
            for j in range(D // 16):
                zrowbuf[i, pl.ds(j * 16, 16)] = zeros16
            return carry
        lax.fori_loop(0, CH, zrow_init, 0)
        zero_accwv_slice()
        pltpu.sync_copy(zrowbuf.at[pl.ds(0, ZROWS_PER_TILE)],
                        accz.at[pl.ds(sid * ZROWS_PER_TILE, ZROWS_PER_TILE)])
        plsc.subcore_barrier()

        def wv_contrib_loop():
            def cgroup(g, gcarry):
                e_ids = lane + g * 16
                rowg = lane * 0 + g
                for h in range(H):
                    sx = plsc.load_gather(sxbuf, [rowg, lane * 8 + h])
                    for d in range(DH):
                        col = jnp.full((16,), h * DH + d, jnp.int32)
                        vd = plsc.load_gather(kbuf, [e_ids, col])
                        plsc.store_scatter(crow, [e_ids, col], vd * sx)
                return gcarry
            lax.fori_loop(0, NG, cgroup, 0)

        def scatter_wv(half):
            def widx(g, gcarry):
                e_ids = lane + g * 16
                dstv = plsc.load_gather(dstbuf, [e_ids])
                ok = (dstv >= half * NH).astype(jnp.int32) * \
                     (dstv < (half + 1) * NH).astype(jnp.int32)
                row = ok * (dstv - half * NH) + (1 - ok) * (NH + e_ids)
                plsc.store_scatter(dstbuf, [e_ids], row)
                return gcarry
            lax.fori_loop(0, NG, widx, 0)
            pltpu.sync_copy(crow, accwv.at[dstbuf], add=True)

        def chunk0(t, carry):
            kchunk = t * NW + wid

            @pl.when(kchunk < NCHUNK)
            def _():
                base = kchunk * CH
                pltpu.sync_copy(src_hbm.at[pl.ds(base, CH)], srcbuf)
                pltpu.sync_copy(dst_hbm.at[pl.ds(base, CH)], dstbuf)
                pltpu.sync_copy(pes_hbm.at[pl.ds(kchunk * CHP, CHP)], pesbuf)
                cp0 = pltpu.async_copy(kh_hbm.at[srcbuf], kbuf, sem0)
                cp1 = pltpu.async_copy(qh_hbm.at[dstbuf], qbuf, sem1)
                cp0.wait()
                cp1.wait()

                def group(g, gcarry):
                    e_ids = lane + g * 16
                    rowg = lane * 0 + g
                    dstv = plsc.load_gather(dstbuf, [e_ids])
                    zoff = (dstv % 16) * 8
                    for h in range(H):
                        accv = zeros16
                        for d in range(DH):
                            col = jnp.full((16,), h * DH + d, jnp.int32)
                            kd = plsc.load_gather(kbuf, [e_ids, col])
                            qd = plsc.load_gather(qbuf, [e_ids, col])
                            accv = accv + kd * qd
                        pcol = lane * 8 + h
                        pes = plsc.load_gather(pesbuf, [rowg, pcol])
                        tt = jnp.minimum(jnp.maximum(accv * 0.25 * pes, -5.0), 5.0)
                        sx = jnp.exp(tt)
                        plsc.store_scatter(srawbuf, [rowg, pcol], accv)
                        plsc.store_scatter(sxbuf, [rowg, pcol], sx)
                        plsc.store_scatter(zrowbuf, [e_ids, zoff + h], sx)
                    return gcarry
                lax.fori_loop(0, NG, group, 0)

                pltpu.sync_copy(srawbuf, sraw_hbm.at[pl.ds(kchunk * CHP, CHP)])
                pltpu.sync_copy(sxbuf, sexp_hbm.at[pl.ds(kchunk * CHP, CHP)])

                cp2 = pltpu.async_copy(vh_hbm.at[srcbuf], kbuf, sem0)
                cp2.wait()
                def zidx(g, gcarry):
                    e_ids = lane + g * 16
                    dstv = plsc.load_gather(dstbuf, [e_ids])
                    plsc.store_scatter(srcbuf, [e_ids], dstv // 16)
                    return gcarry
                lax.fori_loop(0, NG, zidx, 0)

                wv_contrib_loop()
                pltpu.sync_copy(zrowbuf, accz.at[srcbuf], add=True)

                def unz(g, gcarry):
                    e_ids = lane + g * 16
                    dstv = plsc.load_gather(dstbuf, [e_ids])
                    zoff = (dstv % 16) * 8
                    for h in range(H):
                        plsc.store_scatter(zrowbuf, [e_ids, zoff + h], zeros16)
                    return gcarry
                lax.fori_loop(0, NG, unz, 0)

                scatter_wv(0)
            return carry
        lax.fori_loop(0, CHUNKS_PER_W, chunk0, 0)

        plsc.subcore_barrier()
        read_out_accwv(0)
        z0 = sid * ZROWS_PER_TILE
        pltpu.sync_copy(accz.at[pl.ds(z0, ZROWS_PER_TILE)],
                        accz_hbm.at[cid, pl.ds(z0, ZROWS_PER_TILE)])
        zero_accwv_slice()
        plsc.subcore_barrier()

        def chunk1(t, carry):
            kchunk = t * NW + wid

            @pl.when(kchunk < NCHUNK)
            def _():
                base = kchunk * CH
                pltpu.sync_copy(src_hbm.at[pl.ds(base, CH)], srcbuf)
                pltpu.sync_copy(dst_hbm.at[pl.ds(base, CH)], dstbuf)
                pltpu.sync_copy(sexp_hbm.at[pl.ds(kchunk * CHP, CHP)], sxbuf)
                cp2 = pltpu.async_copy(vh_hbm.at[srcbuf], kbuf, sem0)
                cp2.wait()
                wv_contrib_loop()
                scatter_wv(1)
            return carry
        lax.fori_loop(0, CHUNKS_PER_W, chunk1, 0)

        plsc.subcore_barrier()
        read_out_accwv(1)

    return sc_kern(qh, kh, vh, src, dst, pes2)


def _edge_body(e_ref, sraw_ref, we_ref, oew_ref, oeb_ref, g1_ref, b1_ref,
               f1w_ref, f1b_ref, f2w_ref, f2b_ref, g2_ref, b2_ref, out_ref):
    eb = e_ref[...]
    pe = jnp.dot(eb, we_ref[...], preferred_element_type=jnp.float32)
    r = lax.broadcasted_iota(jnp.int32, (H, D), 0)
    c = lax.broadcasted_iota(jnp.int32, (H, D), 1)
    bcast = (c // DH == r).astype(jnp.float32)
    s128 = jnp.dot(sraw_ref[...], bcast, preferred_element_type=jnp.float32)
    score = pe * s128 * 0.25
    ee1 = jnp.dot(score, oew_ref[...], preferred_element_type=jnp.float32)
    ee1 = ee1 + oeb_ref[...] + eb
    ee1 = _ln_tc(ee1, g1_ref[...], b1_ref[...])
    hid = jnp.dot(ee1, f1w_ref[...], preferred_element_type=jnp.float32)
    hid = jnp.maximum(hid + f1b_ref[...], 0.0)
    ef = jnp.dot(hid, f2w_ref[...], preferred_element_type=jnp.float32)
    ef = ef + f2b_ref[...]
    out_ref[...] = _ln_tc(ee1 + ef, g2_ref[...], b2_ref[...])


def _tc_edge(e, sraw, p):
    be = 2000
    w128 = pl.BlockSpec((D, D), lambda i: (0, 0))
    wup = pl.BlockSpec((D, 2 * D), lambda i: (0, 0))
    wdn = pl.BlockSpec((2 * D, D), lambda i: (0, 0))
    v128 = pl.BlockSpec((1, D), lambda i: (0, 0))
    v256 = pl.BlockSpec((1, 2 * D), lambda i: (0, 0))
    r_spec = pl.BlockSpec((be, D), lambda i: (i, 0))
    return pl.pallas_call(
        _edge_body,
        grid=(E // be,),
        in_specs=[r_spec, pl.BlockSpec((be, H), lambda i: (i, 0)),
                  w128, w128, v128, v128, v128,
                  wup, v256, wdn, v128, v128, v128],
        out_specs=r_spec,
        out_shape=jax.ShapeDtypeStruct((E, D), jnp.float32),
    )(e, sraw, p['We'], p['Oe_w'], p['Oe_b'].reshape(1, D),
      p['ln1e_g'].reshape(1, D), p['ln1e_b'].reshape(1, D),
      p['f_e1_w'], p['f_e1_b'].reshape(1, 2 * D),
      p['f_e2_w'], p['f_e2_b'].reshape(1, D),
      p['ln2e_g'].reshape(1, D), p['ln2e_b'].reshape(1, D))


def _node_body(h_ref, acc_ref, zr_ref, ohw_ref, ohb_ref, g1_ref, b1_ref,
               f1w_ref, f1b_ref, f2w_ref, f2b_ref, g2_ref, b2_ref, out_ref):
    a = acc_ref[...]
    wv = a[0, 0] + a[1, 0]
    zr = zr_ref[...]
    zr = zr[0] + zr[1]
    k = lax.broadcasted_iota(jnp.int32, (D, D), 0)
    c = lax.broadcasted_iota(jnp.int32, (D, D), 1)
    parts = []
    for m in range(16):
        sel = (k == m * 8 + c // DH).astype(jnp.float32)
        parts.append(jnp.dot(zr, sel, preferred_element_type=jnp.float32))
    z128 = jnp.stack(parts, axis=1).reshape(wv.shape)
    h_attn = wv / (z128 + 1e-6)
    hb = h_ref[...]
    hh1 = jnp.dot(h_attn, ohw_ref[...], preferred_element_type=jnp.float32)
    hh1 = hh1 + ohb_ref[...] + hb
    hh1 = _ln_tc(hh1, g1_ref[...], b1_ref[...])
    hid = jnp.dot(hh1, f1w_ref[...], preferred_element_type=jnp.float32)
    hid = jnp.maximum(hid + f1b_ref[...], 0.0)
    hf = jnp.dot(hid, f2w_ref[...], preferred_element_type=jnp.float32)
    hf = hf + f2b_ref[...]
    out_ref[...] = _ln_tc(hh1 + hf, g2_ref[...], b2_ref[...])


def _tc_node(hp, acc, zacc, p):
    bn = 1280
    w128 = pl.BlockSpec((D, D), lambda i: (0, 0))
    wup = pl.BlockSpec((D, 2 * D), lambda i: (0, 0))
    wdn = pl.BlockSpec((2 * D, D), lambda i: (0, 0))
    v128 = pl.BlockSpec((1, D), lambda i: (0, 0))
    v256 = pl.BlockSpec((1, 2 * D), lambda i: (0, 0))
    return pl.pallas_call(
        _node_body,
        grid=(NPAD // bn,),
        in_specs=[pl.BlockSpec((bn, D), lambda i: (i, 0)),
                  pl.BlockSpec((NC, 1, bn, D), lambda i: (0, i // 4, i % 4, 0)),
                  pl.BlockSpec((NC, bn // 16, D), lambda i: (0, i, 0)),
                  w128, v128, v128, v128,
                  wup, v256, wdn, v128, v128, v128],
        out_specs=pl.BlockSpec((bn, D), lambda i: (i, 0)),
        out_shape=jax.ShapeDtypeStruct((NPAD, D), jnp.float32),
    )(hp, acc, zacc, p['Oh_w'], p['Oh_b'].reshape(1, D),
      p['ln1h_g'].reshape(1, D), p['ln1h_b'].reshape(1, D),
      p['f_h1_w'], p['f_h1_b'].reshape(1, 2 * D),
      p['f_h2_w'], p['f_h2_b'].reshape(1, D),
      p['ln2h_g'].reshape(1, D), p['ln2h_b'].reshape(1, D))


def kernel(h, e, edge_index, params):
    src = edge_index[0]
    dst = edge_index[1]
    qh, kh, vh = _tc_qkv(h, params['Wq'], params['Wk'], params['Wv'])
    pes2 = _tc_pesum(e, params['We']).reshape(EP, D)
    sraw2, _sexp, acc, zacc = _sc_attention(qh, kh, vh, src, dst, pes2)
    ee = _tc_edge(e, sraw2.reshape(E, H), params)
    hp = jnp.pad(h, ((0, NPAD - N), (0, 0)))
    hh = _tc_node(hp, acc, zacc, params)[:N]
    return hh, ee


# speedup vs baseline: 9.9324x; 1.4283x over previous
import functools

import jax
import jax.numpy as jnp
from jax import lax
from jax.experimental import pallas as pl
from jax.experimental.pallas import tpu as pltpu
from jax.experimental.pallas import tpu_sc as plsc

N = 10000
E = 160000
D = 128
H = 8
DH = D // H

NC = 2
NS = 16
NW = NC * NS
CH = 128
NG = CH // 16
NCHUNK = E // CH
CHUNKS_PER_W = -(-NCHUNK // NW)
NPAD = 10240
ROWS_PER_TILE = NPAD // NS
NZROW = NPAD // 16
ZROWS_PER_TILE = NZROW // NS


def _ln_tc(x, g, b):
    m = jnp.mean(x, axis=-1, keepdims=True)
    v = jnp.mean((x - m) * (x - m), axis=-1, keepdims=True)
    return (x - m) / jnp.sqrt(v + 1e-5) * g + b


def _qkv_body(h_ref, wq_ref, wk_ref, wv_ref, q_ref, k_ref, v_ref):
    hb = h_ref[...]
    q_ref[...] = jnp.dot(hb, wq_ref[...], preferred_element_type=jnp.float32)
    k_ref[...] = jnp.dot(hb, wk_ref[...], preferred_element_type=jnp.float32)
    v_ref[...] = jnp.dot(hb, wv_ref[...], preferred_element_type=jnp.float32)


def _tc_qkv(h, wq, wk, wv):
    bn = 2000
    w_spec = pl.BlockSpec((D, D), lambda i: (0, 0))
    r_spec = pl.BlockSpec((bn, D), lambda i: (i, 0))
    return pl.pallas_call(
        _qkv_body,
        grid=(N // bn,),
        in_specs=[r_spec, w_spec, w_spec, w_spec],
        out_specs=[r_spec, r_spec, r_spec],
        out_shape=[jax.ShapeDtypeStruct((N, D), jnp.float32)] * 3,
    )(h, wq, wk, wv)


def _pesum_body(e_ref, we_ref, ps_ref):
    pe = jnp.dot(e_ref[...], we_ref[...], preferred_element_type=jnp.float32)
    r = lax.broadcasted_iota(jnp.int32, (D, H), 0)
    c = lax.broadcasted_iota(jnp.int32, (D, H), 1)
    sel = (r // DH == c).astype(jnp.float32)
    ps_ref[...] = jnp.dot(pe, sel, preferred_element_type=jnp.float32)


def _tc_pesum(e, we):
    be = 2000
    return pl.pallas_call(
        _pesum_body,
        grid=(E // be,),
        in_specs=[pl.BlockSpec((be, D), lambda i: (i, 0)),
                  pl.BlockSpec((D, D), lambda i: (0, 0))],
        out_specs=pl.BlockSpec((be, H), lambda i: (i, 0)),
        out_shape=jax.ShapeDtypeStruct((E, H), jnp.float32),
    )(e, we)


CHP = CH // 16
EP = E // 16
ZB = 32


def _sc_attention(qh, kh, vh, src, dst, pes2):
    mesh = plsc.VectorSubcoreMesh(
        core_axis_name="c", subcore_axis_name="s",
        num_cores=NC, num_subcores=NS)

    @functools.partial(
        pl.kernel,
        out_type=[
            jax.ShapeDtypeStruct((EP, D), jnp.float32),
            jax.ShapeDtypeStruct((NC, NPAD, D), jnp.float32),
            jax.ShapeDtypeStruct((NC, NZROW, D), jnp.float32),
        ],
        mesh=mesh,
        compiler_params=pltpu.CompilerParams(needs_layout_passes=False),
        scratch_types=[
            pltpu.VMEM((CH, D), jnp.float32),
            pltpu.VMEM((CH, D), jnp.float32),
            pltpu.VMEM((ZB, D), jnp.float32),
            pltpu.VMEM((CHP, D), jnp.float32),
            pltpu.VMEM((CHP, D), jnp.float32),
            pltpu.VMEM((CH,), jnp.int32),
            pltpu.VMEM((CH,), jnp.int32),
            pltpu.VMEM((ZB,), jnp.int32),
            pltpu.VMEM_SHARED((NPAD, D), jnp.float32),
            pltpu.VMEM_SHARED((NZROW, D), jnp.float32),
            pltpu.SemaphoreType.DMA,
            pltpu.SemaphoreType.DMA,
        ],
    )
    def sc_kern(qh_hbm, kh_hbm, vh_hbm, src_hbm, dst_hbm, pes_hbm,
                sraw_hbm, accwv_hbm, accz_hbm,
                kbuf, qbuf, zrowbuf, pesbuf, sxbuf,
                srcbuf, dstbuf, zidxbuf,
                accwv, accz, sem0, sem1):
        cid = lax.axis_index("c")
        sid = lax.axis_index("s")
        wid = sid * NC + cid
        lane = lax.broadcasted_iota(jnp.int32, (16,), 0)
        zeros16 = jnp.zeros((16,), jnp.float32)
        zrow0 = sid * ZROWS_PER_TILE

        def zrow_init(i, carry):
            for j in range(D // 16):
                zrowbuf[i, pl.ds(j * 16, 16)] = zeros16
            return carry
        lax.fori_loop(0, ZB, zrow_init, 0)
        for t in range(ROWS_PER_TILE // ZB):
            pltpu.sync_copy(zrowbuf,
                            accwv.at[pl.ds(sid * ROWS_PER_TILE + t * ZB, ZB)])
        pltpu.sync_copy(zrowbuf, accz.at[pl.ds(zrow0, ZB)])
        pltpu.sync_copy(zrowbuf.at[pl.ds(0, ZROWS_PER_TILE - ZB)],
                        accz.at[pl.ds(zrow0 + ZB, ZROWS_PER_TILE - ZB)])
        plsc.subcore_barrier()

        def chunk_body(t, carry):
            kchunk = t * NW + wid

            @pl.when(kchunk < NCHUNK)
            def _():
                base = kchunk * CH
                pltpu.sync_copy(src_hbm.at[pl.ds(base, CH)], srcbuf)
                pltpu.sync_copy(dst_hbm.at[pl.ds(base, CH)], dstbuf)
                pltpu.sync_copy(pes_hbm.at[pl.ds(kchunk * CHP, CHP)], pesbuf)
                cp0 = pltpu.async_copy(kh_hbm.at[srcbuf], kbuf, sem0)
                cp1 = pltpu.async_copy(qh_hbm.at[dstbuf], qbuf, sem1)
                cp0.wait()
                cp1.wait()

                def score_group(g, zbase):
                    e_ids = lane + g * 16
                    rowg = lane * 0 + g
                    zrows = lane + zbase
                    dstv = plsc.load_gather(dstbuf, [e_ids])
                    plsc.store_scatter(zidxbuf, [zrows], dstv // 16)
                    zoff = (dstv % 16) * 8
                    for h in range(H):
                        accv = zeros16
                        for d in range(DH):
                            col = jnp.full((16,), h * DH + d, jnp.int32)
                            kd = plsc.load_gather(kbuf, [e_ids, col])
                            qd = plsc.load_gather(qbuf, [e_ids, col])
                            accv = accv + kd * qd
                        pcol = lane * 8 + h
                        pes = plsc.load_gather(pesbuf, [rowg, pcol])
                        tt = jnp.minimum(jnp.maximum(accv * 0.25 * pes, -5.0), 5.0)
                        sx = jnp.exp(tt)
                        plsc.store_scatter(pesbuf, [rowg, pcol], accv)
                        plsc.store_scatter(sxbuf, [rowg, pcol], sx)
                        plsc.store_scatter(zrowbuf, [zrows, zoff + h], sx)
                    return zoff

                def pair(p, pcarry):
                    za = score_group(2 * p, 0)
                    zb = score_group(2 * p + 1, 16)
                    pltpu.sync_copy(zrowbuf, accz.at[zidxbuf], add=True)
                    for zrows, zoff in ((lane, za), (lane + 16, zb)):
                        for h in range(H):
                            plsc.store_scatter(zrowbuf, [zrows, zoff + h],
                                               zeros16)
                    return pcarry
                lax.fori_loop(0, NG // 2, pair, 0)

                pltpu.sync_copy(pesbuf, sraw_hbm.at[pl.ds(kchunk * CHP, CHP)])

                cp2 = pltpu.async_copy(vh_hbm.at[srcbuf], kbuf, sem0)
                cp2.wait()

                def cgroup(g, gcarry):
                    e_ids = lane + g * 16
                    rowg = lane * 0 + g
                    for h in range(H):
                        sx = plsc.load_gather(sxbuf, [rowg, lane * 8 + h])
                        for d in range(DH):
                            col = jnp.full((16,), h * DH + d, jnp.int32)
                            vd = plsc.load_gather(kbuf, [e_ids, col])
                            plsc.store_scatter(qbuf, [e_ids, col], vd * sx)
                    return gcarry
                lax.fori_loop(0, NG, cgroup, 0)

                pltpu.sync_copy(qbuf, accwv.at[dstbuf], add=True)
            return carry
        lax.fori_loop(0, CHUNKS_PER_W, chunk_body, 0)

        plsc.subcore_barrier()
        for t in range(ROWS_PER_TILE // CH):
            r0 = sid * ROWS_PER_TILE + t * CH
            pltpu.sync_copy(accwv.at[pl.ds(r0, CH)],
                            accwv_hbm.at[cid, pl.ds(r0, CH)])
        pltpu.sync_copy(accz.at[pl.ds(zrow0, ZROWS_PER_TILE)],
                        accz_hbm.at[cid, pl.ds(zrow0, ZROWS_PER_TILE)])

    return sc_kern(qh, kh, vh, src, dst, pes2)


def _edge_body(e_ref, sraw_ref, we_ref, oew_ref, oeb_ref, g1_ref, b1_ref,
               f1w_ref, f1b_ref, f2w_ref, f2b_ref, g2_ref, b2_ref, out_ref):
    eb = e_ref[...]
    pe = jnp.dot(eb, we_ref[...], preferred_element_type=jnp.float32)
    r = lax.broadcasted_iota(jnp.int32, (H, D), 0)
    c = lax.broadcasted_iota(jnp.int32, (H, D), 1)
    bcast = (c // DH == r).astype(jnp.float32)
    s128 = jnp.dot(sraw_ref[...], bcast, preferred_element_type=jnp.float32)
    score = pe * s128 * 0.25
    ee1 = jnp.dot(score, oew_ref[...], preferred_element_type=jnp.float32)
    ee1 = ee1 + oeb_ref[...] + eb
    ee1 = _ln_tc(ee1, g1_ref[...], b1_ref[...])
    hid = jnp.dot(ee1, f1w_ref[...], preferred_element_type=jnp.float32)
    hid = jnp.maximum(hid + f1b_ref[...], 0.0)
    ef = jnp.dot(hid, f2w_ref[...], preferred_element_type=jnp.float32)
    ef = ef + f2b_ref[...]
    out_ref[...] = _ln_tc(ee1 + ef, g2_ref[...], b2_ref[...])


def _tc_edge(e, sraw, p):
    be = 2000
    w128 = pl.BlockSpec((D, D), lambda i: (0, 0))
    wup = pl.BlockSpec((D, 2 * D), lambda i: (0, 0))
    wdn = pl.BlockSpec((2 * D, D), lambda i: (0, 0))
    v128 = pl.BlockSpec((1, D), lambda i: (0, 0))
    v256 = pl.BlockSpec((1, 2 * D), lambda i: (0, 0))
    r_spec = pl.BlockSpec((be, D), lambda i: (i, 0))
    return pl.pallas_call(
        _edge_body,
        grid=(E // be,),
        in_specs=[r_spec, pl.BlockSpec((be, H), lambda i: (i, 0)),
                  w128, w128, v128, v128, v128,
                  wup, v256, wdn, v128, v128, v128],
        out_specs=r_spec,
        out_shape=jax.ShapeDtypeStruct((E, D), jnp.float32),
    )(e, sraw, p['We'], p['Oe_w'], p['Oe_b'].reshape(1, D),
      p['ln1e_g'].reshape(1, D), p['ln1e_b'].reshape(1, D),
      p['f_e1_w'], p['f_e1_b'].reshape(1, 2 * D),
      p['f_e2_w'], p['f_e2_b'].reshape(1, D),
      p['ln2e_g'].reshape(1, D), p['ln2e_b'].reshape(1, D))


def _node_body(h_ref, acc_ref, zr_ref, ohw_ref, ohb_ref, g1_ref, b1_ref,
               f1w_ref, f1b_ref, f2w_ref, f2b_ref, g2_ref, b2_ref, out_ref):
    a = acc_ref[...]
    wv = a[0] + a[1]
    zr = zr_ref[...]
    zr = zr[0] + zr[1]
    k = lax.broadcasted_iota(jnp.int32, (D, D), 0)
    c = lax.broadcasted_iota(jnp.int32, (D, D), 1)
    parts = []
    for m in range(16):
        sel = (k == m * 8 + c // DH).astype(jnp.float32)
        parts.append(jnp.dot(zr, sel, preferred_element_type=jnp.float32))
    z128 = jnp.stack(parts, axis=1).reshape(wv.shape)
    h_attn = wv / (z128 + 1e-6)
    hb = h_ref[...]
    hh1 = jnp.dot(h_attn, ohw_ref[...], preferred_element_type=jnp.float32)
    hh1 = hh1 + ohb_ref[...] + hb
    hh1 = _ln_tc(hh1, g1_ref[...], b1_ref[...])
    hid = jnp.dot(hh1, f1w_ref[...], preferred_element_type=jnp.float32)
    hid = jnp.maximum(hid + f1b_ref[...], 0.0)
    hf = jnp.dot(hid, f2w_ref[...], preferred_element_type=jnp.float32)
    hf = hf + f2b_ref[...]
    out_ref[...] = _ln_tc(hh1 + hf, g2_ref[...], b2_ref[...])


def _tc_node(hp, acc, zacc, p):
    bn = 1280
    w128 = pl.BlockSpec((D, D), lambda i: (0, 0))
    wup = pl.BlockSpec((D, 2 * D), lambda i: (0, 0))
    wdn = pl.BlockSpec((2 * D, D), lambda i: (0, 0))
    v128 = pl.BlockSpec((1, D), lambda i: (0, 0))
    v256 = pl.BlockSpec((1, 2 * D), lambda i: (0, 0))
    return pl.pallas_call(
        _node_body,
        grid=(NPAD // bn,),
        in_specs=[pl.BlockSpec((bn, D), lambda i: (i, 0)),
                  pl.BlockSpec((NC, bn, D), lambda i: (0, i, 0)),
                  pl.BlockSpec((NC, bn // 16, D), lambda i: (0, i, 0)),
                  w128, v128, v128, v128,
                  wup, v256, wdn, v128, v128, v128],
        out_specs=pl.BlockSpec((bn, D), lambda i: (i, 0)),
        out_shape=jax.ShapeDtypeStruct((NPAD, D), jnp.float32),
    )(hp, acc, zacc, p['Oh_w'], p['Oh_b'].reshape(1, D),
      p['ln1h_g'].reshape(1, D), p['ln1h_b'].reshape(1, D),
      p['f_h1_w'], p['f_h1_b'].reshape(1, 2 * D),
      p['f_h2_w'], p['f_h2_b'].reshape(1, D),
      p['ln2h_g'].reshape(1, D), p['ln2h_b'].reshape(1, D))


def kernel(h, e, edge_index, params):
    src = edge_index[0]
    dst = edge_index[1]
    qh, kh, vh = _tc_qkv(h, params['Wq'], params['Wk'], params['Wv'])
    pes2 = _tc_pesum(e, params['We']).reshape(EP, D)
    sraw2, acc, zacc = _sc_attention(qh, kh, vh, src, dst, pes2)
    ee = _tc_edge(e, sraw2.reshape(E, H), params)
    hp = jnp.pad(h, ((0, NPAD - N), (0, 0)))
    hh = _tc_node(hp, acc, zacc, params)[:N]
    return hh, ee
